# baseline (speedup 1.0000x reference)
import jax
import jax.numpy as jnp
from jax import lax
from jax.experimental import pallas as pl
from jax.experimental.pallas import tpu as pltpu
from jax.experimental.pallas import tpu_sc as plsc

_B = 16384
_K = 64
_N = 1000000
_NC = 2
_NS = 16
_NW = _NC * _NS
_BPW = _B // _NW
_CH = 128
_NCH = _BPW // _CH
_L = 16
_G = 16
_NG = _BPW // _G
_KP = 2 * _K
_NP = _N // 2
_TCOLS = _N // 128
_CPW = (_TCOLS + _NS - 1) // _NS


def _convert_body(ut_h, it_h, tu_h, ti_h, ou_h, oi_h,
                  tc0, tc1, rb0, rb1, tl, semi_, semo0, semo1):
    c = lax.axis_index("c")
    s = lax.axis_index("s")

    def sweep(tab_h, out_h, tail_h):
        def fire(t, tc):
            col = pl.multiple_of(t * 128, 128)
            pltpu.async_copy(tab_h.at[:, pl.ds(col, 128)], tc, semi_)

        def drain_in(tc):
            pltpu.make_async_copy(tab_h.at[:, pl.ds(0, 128)], tc, semi_).wait()

        def transpose(tc, rb):
            for pc in range(4):
                pvec = pc * _L + lax.iota(jnp.int32, _L)
                src = 2 * pvec
                for k in range(_K):
                    kv = jnp.full((_L,), k, jnp.int32)
                    e = plsc.load_gather(tc, [kv, src])
                    o = plsc.load_gather(tc, [kv, src + 1])
                    plsc.store_scatter(rb, [pvec, kv], e)
                    plsc.store_scatter(rb, [pvec, kv + _K], o)

        def put(t, rb, semo):
            base = pl.multiple_of(t * 64, 8)
            pltpu.async_copy(rb, out_h.at[pl.ds(base, 64)], semo)

        def drain_out(rb, semo):
            pltpu.make_async_copy(out_h.at[pl.ds(0, _K)], rb, semo).wait()

        t0 = s * _CPW

        def guard(t):
            return jnp.logical_and(t < _TCOLS, t < t0 + _CPW)

        @pl.when(guard(t0))
        def _():
            fire(t0, tc0)

        def step(i, carry):
            t = t0 + i
            even = lax.rem(i, 2) == 0

            def do(tc_a, tc_b, rb_a, semo_a):
                @pl.when(guard(t + 1))
                def _():
                    fire(t + 1, tc_b)

                drain_in(tc_a)
                transpose(tc_a, rb_a)

                @pl.when(i >= 2)
                def _():
                    drain_out(rb_a, semo_a)

                put(t, rb_a, semo_a)

            @pl.when(jnp.logical_and(guard(t), even))
            def _():
                do(tc0, tc1, rb0, semo0)

            @pl.when(jnp.logical_and(guard(t), jnp.logical_not(even)))
            def _():
                do(tc1, tc0, rb1, semo1)

            return carry

        lax.fori_loop(0, _CPW, step, 0)
        drain_out(rb0, semo0)
        drain_out(rb1, semo1)
        @pl.when(s == _NS - 1)
        def _():
            pltpu.sync_copy(tail_h, tl)
            pltpu.sync_copy(tl, out_h.at[pl.ds(_NP - 32, 32)])

    @pl.when(c == 0)
    def _():
        sweep(ut_h, ou_h, tu_h)

    @pl.when(c == 1)
    def _():
        sweep(it_h, oi_h, ti_h)


def _bias_body(uid_h, iid_h, ub_h, ib_h, out_h, uidx, iidx, ubg, ibg, outv, sem):
    wid = lax.axis_index("s") * _NC + lax.axis_index("c")
    base = wid * _BPW
    for c in range(_NCH):
        pltpu.sync_copy(uid_h.at[pl.ds(base + c * _CH, _CH)], uidx.at[c])
        pltpu.sync_copy(iid_h.at[pl.ds(base + c * _CH, _CH)], iidx.at[c])
    cps = []
    for c in range(_NCH):
        cps.append(pltpu.async_copy(ub_h.at[uidx.at[c]], ubg.at[pl.ds(c * _CH, _CH)], sem))
        cps.append(pltpu.async_copy(ib_h.at[iidx.at[c]], ibg.at[pl.ds(c * _CH, _CH)], sem))
    for cp in cps:
        cp.wait()

    def body(i, carry):
        outv[pl.ds(i * _L, _L)] = ubg[pl.ds(i * _L, _L)] + ibg[pl.ds(i * _L, _L)]
        return carry

    lax.fori_loop(0, _BPW // _L, body, 0)
    pltpu.sync_copy(outv, out_h.at[pl.ds(base, _BPW)])


def _dot_body(uid_h, iid_h, uf_h, if_h, bs_h, out_h,
              uidx, iidx, bsv, ru0, ru1, ri0, ri1, outv, semu, semi_):
    wid = lax.axis_index("s") * _NC + lax.axis_index("c")
    base = wid * _BPW
    pltpu.sync_copy(uid_h.at[pl.ds(base, _BPW)], uidx)
    pltpu.sync_copy(iid_h.at[pl.ds(base, _BPW)], iidx)
    pltpu.sync_copy(bs_h.at[pl.ds(base, _BPW)], bsv)

    def fire(g, ru, ri):
        r0 = g * _G
        uvec = uidx[pl.ds(r0, _G)]
        ivec = iidx[pl.ds(r0, _G)]
        for l in range(_G):
            pltpu.async_copy(uf_h.at[lax.shift_right_logical(uvec[l], 1)], ru.at[l], semu)
            pltpu.async_copy(if_h.at[lax.shift_right_logical(ivec[l], 1)], ri.at[l], semi_)

    def drain(ru, ri):
        pltpu.make_async_copy(uf_h.at[pl.ds(0, _G)], ru, semu).wait()
        pltpu.make_async_copy(if_h.at[pl.ds(0, _G)], ri, semi_).wait()

    lanes = lax.iota(jnp.int32, _L)

    def compute(g, ru, ri):
        r0 = g * _G
        upar = lax.bitwise_and(uidx[pl.ds(r0, _G)], 1) * _K
        ipar = lax.bitwise_and(iidx[pl.ds(r0, _G)], 1) * _K
        acc = bsv[pl.ds(r0, _G)]
        for j in range(_K):
            acc = acc + (plsc.load_gather(ru, [lanes, upar + j])
                         * plsc.load_gather(ri, [lanes, ipar + j]))
        outv[pl.ds(r0, _G)] = acc

    fire(0, ru0, ri0)

    def pair(h, carry):
        g0 = 2 * h
        g1 = g0 + 1

        @pl.when(g1 < _NG)
        def _():
            fire(g1, ru1, ri1)

        drain(ru0, ri0)
        compute(g0, ru0, ri0)

        @pl.when(g0 + 2 < _NG)
        def _():
            fire(g0 + 2, ru0, ri0)

        @pl.when(g1 < _NG)
        def _():
            drain(ru1, ri1)
            compute(g1, ru1, ri1)

        return carry

    lax.fori_loop(0, (_NG + 1) // 2, pair, 0)
    pltpu.sync_copy(outv, out_h.at[pl.ds(base, _BPW)])


def kernel(user_id, item_id, user_factors, item_factors, user_bias, item_bias):
    uid = user_id.reshape(_B)
    iid = item_id.reshape(_B)
    mesh = plsc.VectorSubcoreMesh(core_axis_name="c", subcore_axis_name="s")

    tc_params = pltpu.CompilerParams(
        needs_layout_passes=False, use_tc_tiling_on_sc=True)
    lin_params = pltpu.CompilerParams(
        needs_layout_passes=False, use_tc_tiling_on_sc=False)

    tail_u = user_factors[_N - 64:].reshape(32, _KP)
    tail_i = item_factors[_N - 64:].reshape(32, _KP)

    convert_call = pl.kernel(
        _convert_body,
        out_type=(jax.ShapeDtypeStruct((_NP, _KP), jnp.float32),
                  jax.ShapeDtypeStruct((_NP, _KP), jnp.float32)),
        mesh=mesh,
        scratch_types=[
            pltpu.VMEM((_K, 128), jnp.float32),
            pltpu.VMEM((_K, 128), jnp.float32),
            pltpu.VMEM((_K, _KP), jnp.float32),
            pltpu.VMEM((_K, _KP), jnp.float32),
            pltpu.VMEM((32, _KP), jnp.float32),
            pltpu.SemaphoreType.DMA,
            pltpu.SemaphoreType.DMA,
            pltpu.SemaphoreType.DMA,
        ],
        compiler_params=tc_params,
    )
    su, si = convert_call(user_factors.T, item_factors.T, tail_u, tail_i)

    bias_call = pl.kernel(
        _bias_body,
        out_type=jax.ShapeDtypeStruct((_B,), jnp.float32),
        mesh=mesh,
        scratch_types=[
            pltpu.VMEM((_NCH, _CH), jnp.int32),
            pltpu.VMEM((_NCH, _CH), jnp.int32),
            pltpu.VMEM((_BPW,), jnp.float32),
            pltpu.VMEM((_BPW,), jnp.float32),
            pltpu.VMEM((_BPW,), jnp.float32),
            pltpu.SemaphoreType.DMA,
        ],
        compiler_params=lin_params,
    )
    bias_sum = bias_call(uid, iid, user_bias.reshape(-1), item_bias.reshape(-1))

    dot_call = pl.kernel(
        _dot_body,
        out_type=jax.ShapeDtypeStruct((_B,), jnp.float32),
        mesh=mesh,
        scratch_types=[
            pltpu.VMEM((_BPW,), jnp.int32),
            pltpu.VMEM((_BPW,), jnp.int32),
            pltpu.VMEM((_BPW,), jnp.float32),
            pltpu.VMEM((_G, _KP), jnp.float32),
            pltpu.VMEM((_G, _KP), jnp.float32),
            pltpu.VMEM((_G, _KP), jnp.float32),
            pltpu.VMEM((_G, _KP), jnp.float32),
            pltpu.VMEM((_BPW,), jnp.float32),
            pltpu.SemaphoreType.DMA,
            pltpu.SemaphoreType.DMA,
        ],
        compiler_params=tc_params,
    )
    return dot_call(uid, iid, su, si, bias_sum)


# speedup vs baseline: 3.1701x; 3.1701x over previous
import jax
import jax.numpy as jnp
from jax import lax
from jax.experimental import pallas as pl
from jax.experimental.pallas import tpu as pltpu
from jax.experimental.pallas import tpu_sc as plsc

_B = 16384
_K = 64
_NC = 2
_NS = 16
_NW = _NC * _NS
_BPW = _B // _NW
_CH = 128
_NCH = _BPW // _CH
_L = 16
_G = 16
_NG = _BPW // _G


def _bias_body(uid_h, iid_h, ub_h, ib_h, out_h, uidx, iidx, ubg, ibg, outv, sem):
    wid = lax.axis_index("s") * _NC + lax.axis_index("c")
    base = wid * _BPW
    for c in range(_NCH):
        pltpu.sync_copy(uid_h.at[pl.ds(base + c * _CH, _CH)], uidx.at[c])
        pltpu.sync_copy(iid_h.at[pl.ds(base + c * _CH, _CH)], iidx.at[c])
    cps = []
    for c in range(_NCH):
        cps.append(pltpu.async_copy(ub_h.at[uidx.at[c]], ubg.at[pl.ds(c * _CH, _CH)], sem))
        cps.append(pltpu.async_copy(ib_h.at[iidx.at[c]], ibg.at[pl.ds(c * _CH, _CH)], sem))
    for cp in cps:
        cp.wait()

    def body(i, carry):
        outv[pl.ds(i * _L, _L)] = ubg[pl.ds(i * _L, _L)] + ibg[pl.ds(i * _L, _L)]
        return carry

    lax.fori_loop(0, _BPW // _L, body, 0)
    pltpu.sync_copy(outv, out_h.at[pl.ds(base, _BPW)])


def _dot_body(uid_h, iid_h, uf_h, if_h, bs_h, out_h,
              uidx, iidx, bsv, ru0, ru1, ri0, ri1, outv, semu, semi):
    wid = lax.axis_index("s") * _NC + lax.axis_index("c")
    base = wid * _BPW
    pltpu.sync_copy(uid_h.at[pl.ds(base, _BPW)], uidx)
    pltpu.sync_copy(iid_h.at[pl.ds(base, _BPW)], iidx)
    pltpu.sync_copy(bs_h.at[pl.ds(base, _BPW)], bsv)

    def fire(g, ru, ri):
        r0 = g * _G
        uvec = uidx[pl.ds(r0, _G)]
        ivec = iidx[pl.ds(r0, _G)]
        for l in range(_G):
            pltpu.async_copy(uf_h.at[uvec[l]], ru.at[l], semu)
            pltpu.async_copy(if_h.at[ivec[l]], ri.at[l], semi)

    def drain(ru, ri):
        pltpu.make_async_copy(uf_h.at[pl.ds(0, _G)], ru, semu).wait()
        pltpu.make_async_copy(if_h.at[pl.ds(0, _G)], ri, semi).wait()

    lanes = lax.iota(jnp.int32, _L)

    def compute(g, ru, ri):
        acc = bsv[pl.ds(g * _G, _G)]
        for j in range(_K):
            cols = jnp.full((_L,), j, jnp.int32)
            acc = acc + (plsc.load_gather(ru, [lanes, cols])
                         * plsc.load_gather(ri, [lanes, cols]))
        outv[pl.ds(g * _G, _G)] = acc

    fire(0, ru0, ri0)

    def pair(h, carry):
        g0 = 2 * h
        g1 = g0 + 1

        @pl.when(g1 < _NG)
        def _():
            fire(g1, ru1, ri1)

        drain(ru0, ri0)
        compute(g0, ru0, ri0)

        @pl.when(g0 + 2 < _NG)
        def _():
            fire(g0 + 2, ru0, ri0)

        @pl.when(g1 < _NG)
        def _():
            drain(ru1, ri1)
            compute(g1, ru1, ri1)

        return carry

    lax.fori_loop(0, (_NG + 1) // 2, pair, 0)
    pltpu.sync_copy(outv, out_h.at[pl.ds(base, _BPW)])


def kernel(user_id, item_id, user_factors, item_factors, user_bias, item_bias):
    uid = user_id.reshape(_B)
    iid = item_id.reshape(_B)
    mesh = plsc.VectorSubcoreMesh(core_axis_name="c", subcore_axis_name="s")

    bias_call = pl.kernel(
        _bias_body,
        out_type=jax.ShapeDtypeStruct((_B,), jnp.float32),
        mesh=mesh,
        scratch_types=[
            pltpu.VMEM((_NCH, _CH), jnp.int32),
            pltpu.VMEM((_NCH, _CH), jnp.int32),
            pltpu.VMEM((_BPW,), jnp.float32),
            pltpu.VMEM((_BPW,), jnp.float32),
            pltpu.VMEM((_BPW,), jnp.float32),
            pltpu.SemaphoreType.DMA,
        ],
        compiler_params=pltpu.CompilerParams(
            needs_layout_passes=False, use_tc_tiling_on_sc=False),
    )
    bias_sum = bias_call(uid, iid, user_bias.reshape(-1), item_bias.reshape(-1))

    dot_call = pl.kernel(
        _dot_body,
        out_type=jax.ShapeDtypeStruct((_B,), jnp.float32),
        mesh=mesh,
        scratch_types=[
            pltpu.VMEM((_BPW,), jnp.int32),
            pltpu.VMEM((_BPW,), jnp.int32),
            pltpu.VMEM((_BPW,), jnp.float32),
            pltpu.VMEM((_G, _K), jnp.float32),
            pltpu.VMEM((_G, _K), jnp.float32),
            pltpu.VMEM((_G, _K), jnp.float32),
            pltpu.VMEM((_G, _K), jnp.float32),
            pltpu.VMEM((_BPW,), jnp.float32),
            pltpu.SemaphoreType.DMA,
            pltpu.SemaphoreType.DMA,
        ],
        compiler_params=pltpu.CompilerParams(
            needs_layout_passes=False, use_tc_tiling_on_sc=True),
    )
    return dot_call(uid, iid, user_factors, item_factors, bias_sum)
